# baseline (speedup 1.0000x reference)
import jax
import jax.numpy as jnp
from jax.experimental import pallas as pl
from jax.experimental.pallas import tpu as pltpu

B, N = 4, 512
RNN, INTER = 128, 256


def _encoder_kernel(f_ref, w1_ref, b1_ref, w2_ref, b2_ref, w3_ref, b3_ref,
                    out_ref):
    mm = lambda a, b: jax.lax.dot_general(
        a, b, (((1,), (0,)), ((), ())), preferred_element_type=jnp.float32)

    w23 = mm(w2_ref[...], w3_ref[...])
    b23 = mm(b2_ref[...][None, :], w3_ref[...])

    h1b = mm(f_ref[...].reshape(2 * N, N), w1_ref[...])

    for j in range(2):
        f = f_ref[j]

        deg = jnp.sum(f, axis=0) + 1.0
        dinv = jax.lax.rsqrt(deg)
        dsq = dinv * dinv
        fnt = jnp.transpose(f) * dinv[:, None] * dinv[None, :]

        def s_apply(x):
            z = jax.lax.dot_general(
                fnt, x, (((1,), (0,)), ((), ())),
                preferred_element_type=jnp.float32)
            return z + x * dsq[:, None]

        x1 = s_apply(h1b[j * N:(j + 1) * N]) + b1_ref[...][None, :]
        t1 = s_apply(x1)
        h3 = mm(t1, w23) + b23
        out_ref[j] = s_apply(h3) + b3_ref[...][None, :]


def kernel(flows, W1, b1, W2, b2, W3, b3):
    full = lambda shape: pl.BlockSpec(shape, lambda b: (0,) * len(shape))
    return pl.pallas_call(
        _encoder_kernel,
        grid=(B // 2,),
        in_specs=[
            pl.BlockSpec((2, N, N), lambda b: (b, 0, 0)),
            full((N, RNN)),
            full((RNN,)),
            full((RNN, INTER)),
            full((INTER,)),
            full((INTER, RNN)),
            full((RNN,)),
        ],
        out_specs=pl.BlockSpec((2, N, RNN), lambda b: (b, 0, 0)),
        out_shape=jax.ShapeDtypeStruct((B, N, RNN), jnp.float32),
        compiler_params=pltpu.CompilerParams(dimension_semantics=("parallel",)),
    )(flows, W1, b1, W2, b2, W3, b3)


# speedup vs baseline: 1.1688x; 1.1688x over previous
import jax
import jax.numpy as jnp
from jax.experimental import pallas as pl
from jax.experimental.pallas import tpu as pltpu

B, N = 4, 512
RNN, INTER = 128, 256


def _encoder_kernel(f_ref, w1_ref, b1_ref, w2_ref, b2_ref, w3_ref, b3_ref,
                    out_ref):
    mm = lambda a, b: jax.lax.dot_general(
        a, b, (((1,), (0,)), ((), ())), preferred_element_type=jnp.float32)

    w23 = mm(w2_ref[...], w3_ref[...])
    b23 = mm(b2_ref[...][None, :], w3_ref[...])

    fs, fnts, dsqs, sapps = [], [], [], []
    for j in range(2):
        f = f_ref[j]
        deg = jnp.sum(f, axis=0) + 1.0
        dinv = jax.lax.rsqrt(deg)
        dsq = dinv * dinv
        fnt = jnp.transpose(f) * dinv[:, None] * dinv[None, :]

        def s_apply(x, fnt=fnt, dsq=dsq):
            z = jax.lax.dot_general(
                fnt, x, (((1,), (0,)), ((), ())),
                preferred_element_type=jnp.float32)
            return z + x * dsq[:, None]

        fs.append(f)
        sapps.append(s_apply)

    h1 = [mm(fs[j], w1_ref[...]) for j in range(2)]
    x1 = [sapps[j](h1[j]) + b1_ref[...][None, :] for j in range(2)]
    t1 = [sapps[j](x1[j]) for j in range(2)]
    h3 = [mm(t1[j], w23) + b23 for j in range(2)]
    for j in range(2):
        out_ref[j] = sapps[j](h3[j]) + b3_ref[...][None, :]


def kernel(flows, W1, b1, W2, b2, W3, b3):
    full = lambda shape: pl.BlockSpec(shape, lambda b: (0,) * len(shape))
    return pl.pallas_call(
        _encoder_kernel,
        grid=(B // 2,),
        in_specs=[
            pl.BlockSpec((2, N, N), lambda b: (b, 0, 0)),
            full((N, RNN)),
            full((RNN,)),
            full((RNN, INTER)),
            full((INTER,)),
            full((INTER, RNN)),
            full((RNN,)),
        ],
        out_specs=pl.BlockSpec((2, N, RNN), lambda b: (b, 0, 0)),
        out_shape=jax.ShapeDtypeStruct((B, N, RNN), jnp.float32),
        compiler_params=pltpu.CompilerParams(dimension_semantics=("parallel",)),
    )(flows, W1, b1, W2, b2, W3, b3)


# speedup vs baseline: 1.2377x; 1.0590x over previous
import jax
import jax.numpy as jnp
from jax.experimental import pallas as pl
from jax.experimental.pallas import tpu as pltpu

B, N = 4, 512
RNN, INTER = 128, 256


def _encoder_kernel(f_ref, w1_ref, b1_ref, w2_ref, b2_ref, w3_ref, b3_ref,
                    out_ref):
    mm = lambda a, b: jax.lax.dot_general(
        a, b, (((1,), (0,)), ((), ())), preferred_element_type=jnp.float32)

    w23 = mm(w2_ref[...], w3_ref[...])
    b23 = mm(b2_ref[...][None, :], w3_ref[...])

    fs, fnts, dsqs, sapps = [], [], [], []
    for j in range(2):
        f = f_ref[j]
        deg = jnp.sum(f, axis=0) + 1.0
        dinv = jax.lax.rsqrt(deg)
        dsq = dinv * dinv
        fn = f * dinv[:, None] * dinv[None, :]

        def s_apply(x, fn=fn, dsq=dsq):
            z = jax.lax.dot_general(
                fn, x, (((0,), (0,)), ((), ())),
                preferred_element_type=jnp.float32)
            return z + x * dsq[:, None]

        fs.append(f)
        sapps.append(s_apply)

    h1 = [mm(fs[j], w1_ref[...]) for j in range(2)]
    x1 = [sapps[j](h1[j]) + b1_ref[...][None, :] for j in range(2)]
    t1 = [sapps[j](x1[j]) for j in range(2)]
    h3 = [mm(t1[j], w23) + b23 for j in range(2)]
    for j in range(2):
        out_ref[j] = sapps[j](h3[j]) + b3_ref[...][None, :]


def kernel(flows, W1, b1, W2, b2, W3, b3):
    full = lambda shape: pl.BlockSpec(shape, lambda b: (0,) * len(shape))
    return pl.pallas_call(
        _encoder_kernel,
        grid=(B // 2,),
        in_specs=[
            pl.BlockSpec((2, N, N), lambda b: (b, 0, 0)),
            full((N, RNN)),
            full((RNN,)),
            full((RNN, INTER)),
            full((INTER,)),
            full((INTER, RNN)),
            full((RNN,)),
        ],
        out_specs=pl.BlockSpec((2, N, RNN), lambda b: (b, 0, 0)),
        out_shape=jax.ShapeDtypeStruct((B, N, RNN), jnp.float32),
        compiler_params=pltpu.CompilerParams(dimension_semantics=("parallel",)),
    )(flows, W1, b1, W2, b2, W3, b3)
